# baseline (speedup 1.0000x reference)
import functools

import jax
import jax.numpy as jnp
from jax import lax
from jax.experimental import pallas as pl
from jax.experimental.pallas import tpu as pltpu
from jax.experimental.pallas import tpu_sc as plsc


def _make_sc_agg(n_rows, n_export, e_pad, d, C):
    NC, NS = 2, 16
    NW = NC * NS
    nch = e_pad // (NW * C)
    assert nch * NW * C == e_pad and nch % 3 == 0
    exp_r = 400
    nck = n_export // exp_r
    assert nck * exp_r == n_export
    npass = -(-nck // NS)

    mesh = plsc.VectorSubcoreMesh(core_axis_name="c", subcore_axis_name="s")

    @functools.partial(
        pl.kernel,
        mesh=mesh,
        compiler_params=pltpu.CompilerParams(use_tc_tiling_on_sc=False),
        out_type=[
            jax.ShapeDtypeStruct((NC, n_export, d), jnp.float32),
            jax.ShapeDtypeStruct((NC, n_export, 16), jnp.float32),
        ],
        scratch_types=[
            pltpu.MemorySpace.VMEM_SHARED((n_rows, d), jnp.float32),
            pltpu.MemorySpace.VMEM_SHARED((n_rows, 16), jnp.float32),
            pltpu.MemorySpace.VMEM((3, 2, C), jnp.int32),
            pltpu.MemorySpace.VMEM((3, C), jnp.int32),
            pltpu.MemorySpace.VMEM((3, C, d), jnp.float32),
            pltpu.MemorySpace.VMEM((C, 16), jnp.float32),
            pltpu.SemaphoreType.DMA,
            pltpu.SemaphoreType.DMA,
            pltpu.SemaphoreType.DMA,
            pltpu.SemaphoreType.DMA,
            pltpu.SemaphoreType.DMA,
            pltpu.SemaphoreType.DMA,
            pltpu.SemaphoreType.DMA,
            pltpu.SemaphoreType.DMA,
            pltpu.SemaphoreType.DMA,
            pltpu.SemaphoreType.DMA,
        ],
    )
    def sc_agg(h_hbm, edges_hbm, zf_hbm, zd_hbm, ones_hbm, feat_hbm, deg_hbm,
               acc_f, acc_d, idx2, sidx, rows, ones,
               sem_i, sem_g0, sem_g1, sem_g2,
               sem_s0, sem_s1, sem_s2, sem_d0, sem_d1, sem_d2):
        c = lax.axis_index("c")
        s = lax.axis_index("s")
        w = c * NS + s
        sem_g = (sem_g0, sem_g1, sem_g2)
        sem_s = (sem_s0, sem_s1, sem_s2)
        sem_d = (sem_d0, sem_d1, sem_d2)

        pltpu.sync_copy(ones_hbm, ones)
        for k in range(npass):
            ck = s + NS * k

            @pl.when(ck < nck)
            def _():
                pltpu.sync_copy(zf_hbm, acc_f.at[pl.ds(ck * exp_r, exp_r)])
                pltpu.sync_copy(zd_hbm, acc_d.at[pl.ds(ck * exp_r, exp_r)])

        plsc.subcore_barrier()

        def idx_start(j, b):
            pltpu.async_copy(
                edges_hbm.at[:, pl.ds((w + NW * j) * C, C)], idx2.at[b], sem_i)

        def idx_wait(b):
            pltpu.make_async_copy(
                edges_hbm.at[:, pl.ds(0, C)], idx2.at[b], sem_i).wait()

        def gather_start(b):
            pltpu.async_copy(h_hbm.at[idx2.at[b, 0]], rows.at[b], sem_g[b])

        def gather_wait(b):
            pltpu.make_async_copy(
                h_hbm.at[idx2.at[b, 0]], rows.at[b], sem_g[b]).wait()

        def scatter_start(b):
            for v in range(C // 16):
                sidx[b, pl.ds(16 * v, 16)] = idx2[b, 1, pl.ds(16 * v, 16)]
            pltpu.async_copy(rows.at[b], acc_f.at[sidx.at[b]], sem_s[b],
                             add=True)
            pltpu.async_copy(ones, acc_d.at[sidx.at[b]], sem_d[b], add=True)

        def scatter_drain(b):
            pltpu.make_async_copy(rows.at[b], acc_f.at[sidx.at[b]],
                                  sem_s[b]).wait()
            pltpu.make_async_copy(ones, acc_d.at[sidx.at[b]],
                                  sem_d[b]).wait()

        idx_start(0, 0)

        def body(jj, carry):
            for r in range(3):
                j = 3 * jj + r

                idx_wait(r)

                @pl.when(j >= 3)
                def _():
                    scatter_drain(r)

                gather_start(r)

                @pl.when(j < nch - 1)
                def _():
                    idx_start(j + 1, (r + 1) % 3)

                @pl.when(j >= 1)
                def _():
                    gather_wait((r + 2) % 3)
                    scatter_start((r + 2) % 3)
            return carry

        lax.fori_loop(0, nch // 3, body, 0)
        bl = (nch - 1) % 3
        gather_wait(bl)
        scatter_start(bl)
        scatter_drain((nch - 3) % 3)
        scatter_drain((nch - 2) % 3)
        scatter_drain(bl)
        plsc.subcore_barrier()

        for k in range(npass):
            ck = s + NS * k

            @pl.when(ck < nck)
            def _():
                r0 = ck * exp_r
                pltpu.sync_copy(acc_f.at[pl.ds(r0, exp_r)],
                                feat_hbm.at[c, pl.ds(r0, exp_r)])
                pltpu.sync_copy(acc_d.at[pl.ds(r0, exp_r)],
                                deg_hbm.at[c, pl.ds(r0, exp_r)])

    return sc_agg


def _tc_self_body(h_ref, ws_ref, b_ref, o_ref):
    ws = jnp.mean(ws_ref[...], axis=0)
    bm = jnp.mean(b_ref[...], axis=0)
    o_ref[...] = (
        jnp.dot(h_ref[...], ws, preferred_element_type=jnp.float32)
        + bm[None, :]
    )


def _tc_combine_body(s_ref, f_ref, d_ref, wn_ref, o_ref):
    agg = f_ref[0] + f_ref[1]
    deg = d_ref[0, :, 0:1] + d_ref[1, :, 0:1]
    h_neigh = agg / jnp.maximum(deg, 1.0)
    wn = jnp.mean(wn_ref[...], axis=0)
    o_ref[...] = s_ref[...] + jnp.dot(
        h_neigh, wn, preferred_element_type=jnp.float32)


def _tc_final(h, feat, deg, W_self, W_neigh, b):
    n, in_dim = h.shape
    nh = W_self.shape[0]
    out_dim = W_self.shape[2]
    self_out = pl.pallas_call(
        _tc_self_body,
        out_shape=jax.ShapeDtypeStruct((n, out_dim), jnp.float32),
    )(h, W_self, b)
    return pl.pallas_call(
        _tc_combine_body,
        out_shape=jax.ShapeDtypeStruct((n, out_dim), jnp.float32),
    )(self_out, feat, deg, W_neigh)


def kernel(h, edge_index, W_self, W_neigh, b):
    n, in_dim = h.shape
    e = edge_index.shape[1]
    C = 96
    step = 32 * C
    nch = -(-e // step)
    nch += (-nch) % 3
    e_pad = nch * step
    n_rows = n + 8

    ei = edge_index.astype(jnp.int32)
    if e_pad != e:
        npad = e_pad - e
        pad = jnp.stack(
            [jnp.arange(npad, dtype=jnp.int32) % n,
             jnp.full((npad,), n, jnp.int32)],
            axis=0)
        ei = jnp.concatenate([ei, pad], axis=1)
    zeros_feat = jnp.zeros((400, in_dim), jnp.float32)
    zeros_deg = jnp.zeros((400, 16), jnp.float32)
    ones_c = jnp.ones((C, 16), jnp.float32)
    sc_agg = _make_sc_agg(n_rows, n, e_pad, in_dim, C)
    feat, deg = sc_agg(h, ei, zeros_feat, zeros_deg, ones_c)
    return _tc_final(h, feat, deg, W_self, W_neigh, b)


# speedup vs baseline: 1.0184x; 1.0184x over previous
import functools

import jax
import jax.numpy as jnp
from jax import lax
from jax.experimental import pallas as pl
from jax.experimental.pallas import tpu as pltpu
from jax.experimental.pallas import tpu_sc as plsc


def _make_sc_agg(n_rows, n_export, e_pad, d, C):
    NC, NS = 2, 16
    NW = NC * NS
    epw = e_pad // NW
    nch = epw // C
    assert nch * NW * C == e_pad and nch >= 6
    exp_r = 400
    nck = n_export // exp_r
    assert nck * exp_r == n_export
    npass = -(-nck // NS)

    mesh = plsc.VectorSubcoreMesh(core_axis_name="c", subcore_axis_name="s")

    @functools.partial(
        pl.kernel,
        mesh=mesh,
        compiler_params=pltpu.CompilerParams(use_tc_tiling_on_sc=False),
        out_type=[
            jax.ShapeDtypeStruct((NC, n_export, d), jnp.float32),
            jax.ShapeDtypeStruct((NC, n_export, 16), jnp.float32),
        ],
        scratch_types=[
            pltpu.MemorySpace.VMEM_SHARED((n_rows, d), jnp.float32),
            pltpu.MemorySpace.VMEM_SHARED((n_rows, 16), jnp.float32),
            pltpu.MemorySpace.VMEM((3, 2, C), jnp.int32),
            pltpu.MemorySpace.VMEM((3, C), jnp.int32),
            pltpu.MemorySpace.VMEM((3, C, d), jnp.float32),
            pltpu.MemorySpace.VMEM((C, 16), jnp.float32),
            pltpu.SemaphoreType.DMA,
            pltpu.SemaphoreType.DMA,
            pltpu.SemaphoreType.DMA,
            pltpu.SemaphoreType.DMA,
            pltpu.SemaphoreType.DMA,
            pltpu.SemaphoreType.DMA,
            pltpu.SemaphoreType.DMA,
            pltpu.SemaphoreType.DMA,
            pltpu.SemaphoreType.DMA,
            pltpu.SemaphoreType.DMA,
        ],
    )
    def sc_agg(h_hbm, edges_hbm, zf_hbm, zd_hbm, ones_hbm, feat_hbm, deg_hbm,
               acc_f, acc_d, idx2, sidx, rows, ones,
               sem_i, sem_g0, sem_g1, sem_g2,
               sem_s0, sem_s1, sem_s2, sem_d0, sem_d1, sem_d2):
        c = lax.axis_index("c")
        s = lax.axis_index("s")
        w = c * NS + s
        sem_g = (sem_g0, sem_g1, sem_g2)
        sem_s = (sem_s0, sem_s1, sem_s2)
        sem_d = (sem_d0, sem_d1, sem_d2)

        pltpu.sync_copy(ones_hbm, ones)
        for k in range(npass):
            ck = s + NS * k

            @pl.when(ck < nck)
            def _():
                pltpu.sync_copy(zf_hbm, acc_f.at[pl.ds(ck * exp_r, exp_r)])
                pltpu.sync_copy(zd_hbm, acc_d.at[pl.ds(ck * exp_r, exp_r)])

        plsc.subcore_barrier()

        def idx_start(j, b):
            pltpu.async_copy(
                edges_hbm.at[:, pl.ds(w * epw + j * C, C)], idx2.at[b], sem_i)

        def idx_wait(b):
            pltpu.make_async_copy(
                edges_hbm.at[:, pl.ds(0, C)], idx2.at[b], sem_i).wait()

        def gather_start(b):
            pltpu.async_copy(h_hbm.at[idx2.at[b, 0]], rows.at[b], sem_g[b])

        def gather_wait(b):
            pltpu.make_async_copy(
                h_hbm.at[idx2.at[b, 0]], rows.at[b], sem_g[b]).wait()

        def scatter_start(b):
            for v in range(C // 16):
                sidx[b, pl.ds(16 * v, 16)] = idx2[b, 1, pl.ds(16 * v, 16)]
            pltpu.async_copy(rows.at[b], acc_f.at[sidx.at[b]], sem_s[b],
                             add=True)
            pltpu.async_copy(ones, acc_d.at[sidx.at[b]], sem_d[b], add=True)

        def scatter_drain(b):
            pltpu.make_async_copy(rows.at[b], acc_f.at[sidx.at[b]],
                                  sem_s[b]).wait()
            pltpu.make_async_copy(ones, acc_d.at[sidx.at[b]],
                                  sem_d[b]).wait()

        idx_start(0, 0)

        def body(jj, carry):
            for r in range(3):
                j = 3 * jj + r

                idx_wait(r)

                @pl.when(j >= 3)
                def _():
                    scatter_drain(r)

                gather_start(r)

                @pl.when(j < nch - 1)
                def _():
                    idx_start(j + 1, (r + 1) % 3)

                @pl.when(j >= 1)
                def _():
                    gather_wait((r + 2) % 3)
                    scatter_start((r + 2) % 3)
            return carry

        lax.fori_loop(0, nch // 3, body, 0)
        for t in range(nch - nch // 3 * 3):
            j = nch // 3 * 3 + t
            idx_wait(t)
            scatter_drain(t)
            gather_start(t)
            if j < nch - 1:
                idx_start(j + 1, (t + 1) % 3)
            gather_wait((t + 2) % 3)
            scatter_start((t + 2) % 3)
        bl = (nch - 1) % 3
        gather_wait(bl)
        scatter_start(bl)
        scatter_drain((nch - 3) % 3)
        scatter_drain((nch - 2) % 3)
        scatter_drain(bl)
        plsc.subcore_barrier()

        for k in range(npass):
            ck = s + NS * k

            @pl.when(ck < nck)
            def _():
                r0 = ck * exp_r
                pltpu.sync_copy(acc_f.at[pl.ds(r0, exp_r)],
                                feat_hbm.at[c, pl.ds(r0, exp_r)])
                pltpu.sync_copy(acc_d.at[pl.ds(r0, exp_r)],
                                deg_hbm.at[c, pl.ds(r0, exp_r)])

    return sc_agg


def _tc_body(h_ref, f_ref, d_ref, ws_ref, wn_ref, b_ref, o_ref):
    agg = f_ref[0] + f_ref[1]
    deg = d_ref[0, :, 0:1] + d_ref[1, :, 0:1]
    h_neigh = agg / jnp.maximum(deg, 1.0)
    ws = jnp.mean(ws_ref[...], axis=0)
    wn = jnp.mean(wn_ref[...], axis=0)
    bm = jnp.mean(b_ref[...], axis=0)
    o_ref[...] = (
        jnp.dot(h_ref[...], ws, preferred_element_type=jnp.float32)
        + jnp.dot(h_neigh, wn, preferred_element_type=jnp.float32)
        + bm[None, :]
    )


def _tc_final(h, feat, deg, W_self, W_neigh, b):
    n, in_dim = h.shape
    nh = W_self.shape[0]
    out_dim = W_self.shape[2]
    R = n
    grid = (n // R,)
    return pl.pallas_call(
        _tc_body,
        grid=grid,
        in_specs=[
            pl.BlockSpec((R, in_dim), lambda i: (i, 0)),
            pl.BlockSpec((2, R, in_dim), lambda i: (0, i, 0)),
            pl.BlockSpec((2, R, 16), lambda i: (0, i, 0)),
            pl.BlockSpec((nh, in_dim, out_dim), lambda i: (0, 0, 0)),
            pl.BlockSpec((nh, in_dim, out_dim), lambda i: (0, 0, 0)),
            pl.BlockSpec((nh, out_dim), lambda i: (0, 0)),
        ],
        out_specs=pl.BlockSpec((R, out_dim), lambda i: (i, 0)),
        out_shape=jax.ShapeDtypeStruct((n, out_dim), jnp.float32),
    )(h, feat, deg, W_self, W_neigh, b)


def kernel(h, edge_index, W_self, W_neigh, b):
    n, in_dim = h.shape
    e = edge_index.shape[1]
    C = 80
    step = 32 * C
    e_pad = -(-e // step) * step
    n_rows = n + 8

    ei = edge_index.astype(jnp.int32)
    if e_pad != e:
        npad = e_pad - e
        pad = jnp.stack(
            [jnp.arange(npad, dtype=jnp.int32) % n,
             jnp.full((npad,), n, jnp.int32)],
            axis=0)
        ei = jnp.concatenate([ei, pad], axis=1)
    zeros_feat = jnp.zeros((400, in_dim), jnp.float32)
    zeros_deg = jnp.zeros((400, 16), jnp.float32)
    ones_c = jnp.ones((C, 16), jnp.float32)
    sc_agg = _make_sc_agg(n_rows, n, e_pad, in_dim, C)
    feat, deg = sc_agg(h, ei, zeros_feat, zeros_deg, ones_c)
    return _tc_final(h, feat, deg, W_self, W_neigh, b)


# speedup vs baseline: 1.0219x; 1.0035x over previous
import functools

import jax
import jax.numpy as jnp
from jax import lax
from jax.experimental import pallas as pl
from jax.experimental.pallas import tpu as pltpu
from jax.experimental.pallas import tpu_sc as plsc


def _make_sc_agg(n_rows, n_export, e_pad, d, C):
    NC, NS = 2, 16
    NW = NC * NS
    epw = e_pad // NW
    nch = epw // C
    assert nch * NW * C == e_pad and nch >= 6
    exp_r = 400
    nck = n_export // exp_r
    assert nck * exp_r == n_export
    npass = -(-nck // NS)

    mesh = plsc.VectorSubcoreMesh(core_axis_name="c", subcore_axis_name="s")

    @functools.partial(
        pl.kernel,
        mesh=mesh,
        compiler_params=pltpu.CompilerParams(use_tc_tiling_on_sc=False),
        out_type=[
            jax.ShapeDtypeStruct((NC, n_export, d), jnp.float32),
            jax.ShapeDtypeStruct((NC, n_export, 16), jnp.float32),
        ],
        scratch_types=[
            pltpu.MemorySpace.VMEM_SHARED((n_rows, d), jnp.float32),
            pltpu.MemorySpace.VMEM_SHARED((n_rows, 16), jnp.float32),
            pltpu.MemorySpace.VMEM((3, 2, C), jnp.int32),
            pltpu.MemorySpace.VMEM((3, C), jnp.int32),
            pltpu.MemorySpace.VMEM((3, C, d), jnp.float32),
            pltpu.MemorySpace.VMEM((C, 16), jnp.float32),
            pltpu.SemaphoreType.DMA,
            pltpu.SemaphoreType.DMA,
            pltpu.SemaphoreType.DMA,
            pltpu.SemaphoreType.DMA,
            pltpu.SemaphoreType.DMA,
            pltpu.SemaphoreType.DMA,
            pltpu.SemaphoreType.DMA,
            pltpu.SemaphoreType.DMA,
            pltpu.SemaphoreType.DMA,
            pltpu.SemaphoreType.DMA,
        ],
    )
    def sc_agg(h_hbm, edges_hbm, zf_hbm, zd_hbm, ones_hbm, feat_hbm, deg_hbm,
               acc_f, acc_d, idx2, sidx, rows, ones,
               sem_i, sem_g0, sem_g1, sem_g2,
               sem_s0, sem_s1, sem_s2, sem_d0, sem_d1, sem_d2):
        c = lax.axis_index("c")
        s = lax.axis_index("s")
        w = c * NS + s
        sem_g = (sem_g0, sem_g1, sem_g2)
        sem_s = (sem_s0, sem_s1, sem_s2)
        sem_d = (sem_d0, sem_d1, sem_d2)

        pltpu.sync_copy(ones_hbm, ones)
        for k in range(npass):
            ck = s + NS * k

            @pl.when(ck < nck)
            def _():
                pltpu.sync_copy(zf_hbm, acc_f.at[pl.ds(ck * exp_r, exp_r)])
                pltpu.sync_copy(zd_hbm, acc_d.at[pl.ds(ck * exp_r, exp_r)])

        plsc.subcore_barrier()

        def idx_start(j, b):
            pltpu.async_copy(
                edges_hbm.at[:, pl.ds(w * epw + j * C, C)], idx2.at[b], sem_i)

        def idx_wait(b):
            pltpu.make_async_copy(
                edges_hbm.at[:, pl.ds(0, C)], idx2.at[b], sem_i).wait()

        def gather_start(b):
            pltpu.async_copy(h_hbm.at[idx2.at[b, 0]], rows.at[b], sem_g[b])

        def gather_wait(b):
            pltpu.make_async_copy(
                h_hbm.at[idx2.at[b, 0]], rows.at[b], sem_g[b]).wait()

        def scatter_start(b):
            for v in range(C // 16):
                sidx[b, pl.ds(16 * v, 16)] = idx2[b, 1, pl.ds(16 * v, 16)]
            pltpu.async_copy(rows.at[b], acc_f.at[sidx.at[b]], sem_s[b],
                             add=True)
            pltpu.async_copy(ones, acc_d.at[sidx.at[b]], sem_d[b], add=True)

        def scatter_drain(b):
            pltpu.make_async_copy(rows.at[b], acc_f.at[sidx.at[b]],
                                  sem_s[b]).wait()
            pltpu.make_async_copy(ones, acc_d.at[sidx.at[b]],
                                  sem_d[b]).wait()

        idx_start(0, 0)

        def body(jj, carry):
            for r in range(3):
                j = 3 * jj + r

                idx_wait(r)

                @pl.when(j >= 3)
                def _():
                    scatter_drain(r)

                gather_start(r)

                @pl.when(j < nch - 1)
                def _():
                    idx_start(j + 1, (r + 1) % 3)

                @pl.when(j >= 1)
                def _():
                    gather_wait((r + 2) % 3)
                    scatter_start((r + 2) % 3)
            return carry

        lax.fori_loop(0, nch // 3, body, 0)
        for t in range(nch - nch // 3 * 3):
            j = nch // 3 * 3 + t
            idx_wait(t)
            scatter_drain(t)
            gather_start(t)
            if j < nch - 1:
                idx_start(j + 1, (t + 1) % 3)
            gather_wait((t + 2) % 3)
            scatter_start((t + 2) % 3)
        bl = (nch - 1) % 3
        gather_wait(bl)
        scatter_start(bl)
        scatter_drain((nch - 3) % 3)
        scatter_drain((nch - 2) % 3)
        scatter_drain(bl)
        plsc.subcore_barrier()

        for k in range(npass):
            ck = s + NS * k

            @pl.when(ck < nck)
            def _():
                r0 = ck * exp_r
                pltpu.sync_copy(acc_f.at[pl.ds(r0, exp_r)],
                                feat_hbm.at[c, pl.ds(r0, exp_r)])
                pltpu.sync_copy(acc_d.at[pl.ds(r0, exp_r)],
                                deg_hbm.at[c, pl.ds(r0, exp_r)])

    return sc_agg


def _tc_body(h_ref, f_ref, d_ref, ws_ref, wn_ref, b_ref, o_ref):
    agg = f_ref[0] + f_ref[1]
    deg = d_ref[0, :, 0:1] + d_ref[1, :, 0:1]
    h_neigh = agg / jnp.maximum(deg, 1.0)
    ws = jnp.mean(ws_ref[...], axis=0)
    wn = jnp.mean(wn_ref[...], axis=0)
    bm = jnp.mean(b_ref[...], axis=0)
    o_ref[...] = (
        jnp.dot(h_ref[...], ws, preferred_element_type=jnp.float32)
        + jnp.dot(h_neigh, wn, preferred_element_type=jnp.float32)
        + bm[None, :]
    )


def _tc_final(h, feat, deg, W_self, W_neigh, b):
    n, in_dim = h.shape
    nh = W_self.shape[0]
    out_dim = W_self.shape[2]
    R = 2000
    assert n % R == 0
    grid = (n // R,)
    return pl.pallas_call(
        _tc_body,
        grid=grid,
        in_specs=[
            pl.BlockSpec((R, in_dim), lambda i: (i, 0)),
            pl.BlockSpec((2, R, in_dim), lambda i: (0, i, 0)),
            pl.BlockSpec((2, R, 16), lambda i: (0, i, 0)),
            pl.BlockSpec((nh, in_dim, out_dim), lambda i: (0, 0, 0)),
            pl.BlockSpec((nh, in_dim, out_dim), lambda i: (0, 0, 0)),
            pl.BlockSpec((nh, out_dim), lambda i: (0, 0)),
        ],
        out_specs=pl.BlockSpec((R, out_dim), lambda i: (i, 0)),
        out_shape=jax.ShapeDtypeStruct((n, out_dim), jnp.float32),
    )(h, feat, deg, W_self, W_neigh, b)


def kernel(h, edge_index, W_self, W_neigh, b):
    n, in_dim = h.shape
    e = edge_index.shape[1]
    C = 80
    step = 32 * C
    e_pad = -(-e // step) * step
    n_rows = n + 8

    ei = edge_index.astype(jnp.int32)
    if e_pad != e:
        npad = e_pad - e
        pad = jnp.stack(
            [jnp.arange(npad, dtype=jnp.int32) % n,
             jnp.full((npad,), n, jnp.int32)],
            axis=0)
        ei = jnp.concatenate([ei, pad], axis=1)
    zeros_feat = jnp.zeros((400, in_dim), jnp.float32)
    zeros_deg = jnp.zeros((400, 16), jnp.float32)
    ones_c = jnp.ones((C, 16), jnp.float32)
    sc_agg = _make_sc_agg(n_rows, n, e_pad, in_dim, C)
    feat, deg = sc_agg(h, ei, zeros_feat, zeros_deg, ones_c)
    return _tc_final(h, feat, deg, W_self, W_neigh, b)
